# Initial kernel scaffold; baseline (speedup 1.0000x reference)
import jax
import jax.numpy as jnp
from jax.experimental import pallas as pl


def kernel(x, adj_indices, adj_values, W):
    raise NotImplementedError("write your pallas kernel here")


# speedup vs baseline: 8.3853x; 8.3853x over previous
import functools

import jax
import jax.numpy as jnp
from jax import lax
from jax.experimental import pallas as pl
from jax.experimental.pallas import tpu as pltpu
from jax.experimental.pallas import tpu_sc as plsc

_L = 16


def _lane_bcast(v, j):
    dn = lax.GatherDimensionNumbers(
        offset_dims=(), collapsed_slice_dims=(0,), start_index_map=(0,))
    idx = jnp.full((_L, 1), j, jnp.int32)
    return lax.gather(v, idx, dn, (1,),
                      mode=lax.GatherScatterMode.PROMISE_IN_BOUNDS)


def _matmul(x, w):
    n, d = x.shape
    out_f = w.shape[1]

    def body(x_ref, w_ref, h_ref):
        h_ref[...] = jnp.dot(x_ref[...], w_ref[...],
                             preferred_element_type=jnp.float32)

    return pl.pallas_call(
        body,
        out_shape=jax.ShapeDtypeStruct((n, out_f), jnp.float32),
    )(x, w)


def _combine_relu(parts, n):
    _, n_pad, out_f = parts.shape

    def body(p_ref, o_ref):
        s = jnp.maximum(p_ref[0] + p_ref[1], 0.0)
        o_ref[...] = lax.slice(s, (0, 0), (n, out_f))

    return pl.pallas_call(
        body,
        out_shape=jax.ShapeDtypeStruct((n, out_f), jnp.float32),
    )(parts)


def _sc_scatter(h, row2d, col2d, valp, *, nc, ns, cb, sub, g_chunks):
    n, out_f = h.shape
    nw = nc * ns
    nsub = cb // sub
    n_pad = -(-n // (8 * ns)) * (8 * ns)
    rpt = n_pad // ns
    halves = out_f // _L
    ew_subs = g_chunks * nsub
    mesh = plsc.VectorSubcoreMesh(core_axis_name="c", subcore_axis_name="s")

    @functools.partial(
        pl.kernel,
        out_type=jax.ShapeDtypeStruct((nc, n_pad, out_f), jnp.float32),
        mesh=mesh,
        compiler_params=pltpu.CompilerParams(use_tc_tiling_on_sc=False),
        scratch_types=[
            pltpu.VMEM((nsub, sub), jnp.int32),
            pltpu.VMEM((nsub, sub), jnp.int32),
            pltpu.VMEM((cb,), jnp.float32),
            pltpu.VMEM((cb, out_f), jnp.float32),
            pltpu.VMEM_SHARED((n_pad, out_f), jnp.float32),
            pltpu.SemaphoreType.DMA,
        ],
    )
    def k(h_hbm, row_hbm, col_hbm, val_hbm, out_hbm,
          colv, rowv, valv, rowsb, acc, gsem):
        cid = lax.axis_index("c")
        sid = lax.axis_index("s")
        wid = sid * nc + cid
        iota = lax.iota(jnp.int32, _L)
        z16 = jnp.zeros((_L,), jnp.float32)

        def zbody(i, c):
            for hh in range(halves):
                rowsb[i, pl.ds(hh * _L, _L)] = z16
            return c
        lax.fori_loop(0, rpt, zbody, 0)
        pltpu.sync_copy(rowsb.at[pl.ds(0, rpt)],
                        acc.at[pl.ds(sid * rpt, rpt)])
        plsc.subcore_barrier()

        for g in range(g_chunks):
            base_rows = wid * ew_subs + g * nsub
            pltpu.sync_copy(col_hbm.at[pl.ds(base_rows, nsub)], colv)
            pltpu.sync_copy(row_hbm.at[pl.ds(base_rows, nsub)], rowv)
            pltpu.sync_copy(val_hbm.at[pl.ds(base_rows * sub, cb)], valv)

            cps = [pltpu.async_copy(h_hbm.at[colv.at[s]],
                                    rowsb.at[pl.ds(s * sub, sub)], gsem)
                   for s in range(nsub)]
            for cp in cps:
                cp.wait()

            def mbody(gi, c):
                e0 = gi * _L
                v16 = valv[pl.ds(e0, _L)]
                for j in range(_L):
                    vb = _lane_bcast(v16, j)
                    for hh in range(halves):
                        sl = pl.ds(hh * _L, _L)
                        rowsb[e0 + j, sl] = rowsb[e0 + j, sl] * vb
                return c
            lax.fori_loop(0, cb // _L, mbody, 0)

            for s in range(nsub):
                pltpu.sync_copy(rowsb.at[pl.ds(s * sub, sub)],
                                acc.at[rowv.at[s]], add=True)

        plsc.subcore_barrier()
        pltpu.sync_copy(acc.at[pl.ds(sid * rpt, rpt)],
                        out_hbm.at[cid, pl.ds(sid * rpt, rpt)])

    return k(h, row2d, col2d, valp)


def kernel(x, adj_indices, adj_values, W):
    n, _ = x.shape
    e = adj_values.shape[0]
    info = plsc.get_sparse_core_info()
    nc, ns = info.num_cores, info.num_subcores
    nw = nc * ns
    cb = 1024
    sub = 128
    chunk = nw * cb
    g_chunks = -(-e // chunk)
    epad = g_chunks * chunk

    row = adj_indices[0]
    col = adj_indices[1]
    pad = epad - e
    if pad:
        row = jnp.concatenate([row, jnp.zeros((pad,), row.dtype)])
        col = jnp.concatenate([col, jnp.zeros((pad,), col.dtype)])
        vals = jnp.concatenate([adj_values,
                                jnp.zeros((pad,), adj_values.dtype)])
    else:
        vals = adj_values
    row2d = row.reshape(epad // sub, sub)
    col2d = col.reshape(epad // sub, sub)

    h = _matmul(x, W)
    parts = _sc_scatter(h, row2d, col2d, vals,
                        nc=nc, ns=ns, cb=cb, sub=sub, g_chunks=g_chunks)
    return _combine_relu(parts, n)
